# Initial kernel scaffold; baseline (speedup 1.0000x reference)
import jax
import jax.numpy as jnp
from jax.experimental import pallas as pl


def kernel(features, edge_index, edge_types, W1, Wself1, b1, W2, Wself2, b2):
    raise NotImplementedError("write your pallas kernel here")


# speedup vs baseline: 9.2832x; 9.2832x over previous
import jax
import jax.numpy as jnp
from jax import lax
from jax.experimental import pallas as pl
from jax.experimental.pallas import tpu as pltpu
from jax.experimental.pallas import tpu_sc as plsc

N = 10000
E = 320000
R = 8
D = 128

NC = 2
NS = 16
NW = NC * NS
CHUNK = 128
CPW = 80
EPAD = NW * CPW * CHUNK
NPAD = 10112
RPT = NPAD // NS
GRP = 8

_MESH = plsc.VectorSubcoreMesh(core_axis_name="c", subcore_axis_name="s")


def _fill(ref, val16, nrows, width):
    def row(r, _):
        for k in range(width // 16):
            ref[r, pl.ds(k * 16, 16)] = val16
        return 0

    lax.fori_loop(0, nrows, row, 0)


def _zero_acc(acc_sh, stage, s):
    for t in range(5):
        sz = CHUNK if t < 4 else RPT - 4 * CHUNK
        pltpu.sync_copy(stage.at[pl.ds(0, sz)],
                        acc_sh.at[pl.ds(s * RPT + t * CHUNK, sz)])


def _writeback(acc_sh, stage, out, c, s):
    for t in range(5):
        sz = CHUNK if t < 4 else RPT - 4 * CHUNK
        pltpu.sync_copy(acc_sh.at[pl.ds(s * RPT + t * CHUNK, sz)],
                        stage.at[pl.ds(0, sz)])
        pltpu.sync_copy(stage.at[pl.ds(0, sz)],
                        out.at[pl.ds(c * NPAD + s * RPT + t * CHUNK, sz)])


def _sc_scatter(table, src2d, dst2d, et2d):
    scratch = (
        pltpu.VMEM_SHARED((NPAD, D), jnp.float32),
        pltpu.VMEM((GRP, CHUNK), jnp.int32),
        pltpu.VMEM((GRP, CHUNK), jnp.int32),
        pltpu.VMEM((GRP, CHUNK), jnp.int32),
        pltpu.VMEM((CHUNK,), jnp.int32),
        pltpu.VMEM((CHUNK, D), jnp.float32),
        pltpu.SemaphoreType.DMA,
    )

    def body(table_hbm, src_hbm, dst_hbm, et_hbm, acc_out,
             acc_sh, src_v, dst_v, et_v, gkey_v, rows_v, sem):
        c = lax.axis_index("c")
        s = lax.axis_index("s")
        w = c * NS + s

        _fill(rows_v, jnp.zeros((16,), jnp.float32), CHUNK, D)
        _zero_acc(acc_sh, rows_v, s)
        plsc.subcore_barrier()

        def group(g, _):
            base = w * CPW + g * GRP
            pltpu.sync_copy(src_hbm.at[pl.ds(base, GRP)], src_v)
            pltpu.sync_copy(dst_hbm.at[pl.ds(base, GRP)], dst_v)
            pltpu.sync_copy(et_hbm.at[pl.ds(base, GRP)], et_v)
            for j in range(GRP):
                for i in range(CHUNK // 16):
                    et16 = et_v[j, pl.ds(i * 16, 16)]
                    src16 = src_v[j, pl.ds(i * 16, 16)]
                    gkey_v[pl.ds(i * 16, 16)] = et16 * N + src16
                pltpu.async_copy(table_hbm.at[gkey_v], rows_v, sem).wait()
                pltpu.sync_copy(rows_v, acc_sh.at[dst_v.at[j]], add=True)
            return 0

        lax.fori_loop(0, CPW // GRP, group, 0)
        plsc.subcore_barrier()
        _writeback(acc_sh, rows_v, acc_out, c, s)

    out = pl.kernel(body,
                    out_type=jax.ShapeDtypeStruct((NC * NPAD, D), jnp.float32),
                    mesh=_MESH, scratch_types=scratch)(table, src2d, dst2d, et2d)
    return out.reshape(NC, NPAD, D)


def _sc_deg(dst2d):
    scratch = (
        pltpu.VMEM_SHARED((NPAD, D), jnp.float32),
        pltpu.VMEM((GRP, CHUNK), jnp.int32),
        pltpu.VMEM((CHUNK, D), jnp.float32),
    )

    def body(dst_hbm, deg_out, deg_sh, dst_v, ones_v):
        c = lax.axis_index("c")
        s = lax.axis_index("s")
        w = c * NS + s

        _fill(ones_v, jnp.zeros((16,), jnp.float32), CHUNK, D)
        _zero_acc(deg_sh, ones_v, s)
        _fill(ones_v, jnp.ones((16,), jnp.float32), CHUNK, D)
        plsc.subcore_barrier()

        def group(g, _):
            base = w * CPW + g * GRP
            pltpu.sync_copy(dst_hbm.at[pl.ds(base, GRP)], dst_v)
            for j in range(GRP):
                pltpu.sync_copy(ones_v, deg_sh.at[dst_v.at[j]], add=True)
            return 0

        lax.fori_loop(0, CPW // GRP, group, 0)
        plsc.subcore_barrier()
        _writeback(deg_sh, ones_v, deg_out, c, s)

    out = pl.kernel(body,
                    out_type=jax.ShapeDtypeStruct((NC * NPAD, D), jnp.float32),
                    mesh=_MESH, scratch_types=scratch)(dst2d)
    return out.reshape(NC, NPAD, D)


ROWBLK = 2000


def _mm1_body(x_ref, w_ref, o_ref):
    o_ref[0] = jnp.dot(x_ref[...], w_ref[0], preferred_element_type=jnp.float32)


def _tc_mm1(x, wc):
    grid = (wc.shape[0], N // ROWBLK)
    return pl.pallas_call(
        _mm1_body,
        grid=grid,
        in_specs=[
            pl.BlockSpec((ROWBLK, D), lambda j, i: (i, 0)),
            pl.BlockSpec((1, D, D), lambda j, i: (j, 0, 0)),
        ],
        out_specs=pl.BlockSpec((1, ROWBLK, D), lambda j, i: (j, i, 0)),
        out_shape=jax.ShapeDtypeStruct((wc.shape[0], N, D), jnp.float32),
    )(x, wc)


def _norm(acc_ref, deg_ref):
    a = acc_ref[0] + acc_ref[1]
    dg = deg_ref[0, :, 0:1] + deg_ref[1, :, 0:1]
    inv = 1.0 / jnp.maximum(dg, 1.0)
    return a * inv


def _mm2_body(acc_ref, deg_ref, self_ref, b_ref, w_ref, o_ref):
    h = _norm(acc_ref, deg_ref) + self_ref[0] + b_ref[...]
    h = jnp.maximum(h, 0.0)
    o_ref[0] = jnp.dot(h, w_ref[0], preferred_element_type=jnp.float32)


def _tc_mm2(acc, deg, selft, b, wc):
    grid = (wc.shape[0], N // ROWBLK)
    return pl.pallas_call(
        _mm2_body,
        grid=grid,
        in_specs=[
            pl.BlockSpec((NC, ROWBLK, D), lambda j, i: (0, i, 0)),
            pl.BlockSpec((NC, ROWBLK, D), lambda j, i: (0, i, 0)),
            pl.BlockSpec((1, ROWBLK, D), lambda j, i: (0, i, 0)),
            pl.BlockSpec((1, D), lambda j, i: (0, 0)),
            pl.BlockSpec((1, D, D), lambda j, i: (j, 0, 0)),
        ],
        out_specs=pl.BlockSpec((1, ROWBLK, D), lambda j, i: (j, i, 0)),
        out_shape=jax.ShapeDtypeStruct((wc.shape[0], N, D), jnp.float32),
    )(acc, deg, selft, b, wc)


def _final_body(acc_ref, deg_ref, self_ref, b_ref, o_ref):
    o_ref[...] = _norm(acc_ref, deg_ref) + self_ref[0] + b_ref[...]


def _tc_final(acc2, deg, selft, b):
    grid = (N // ROWBLK,)
    return pl.pallas_call(
        _final_body,
        grid=grid,
        in_specs=[
            pl.BlockSpec((NC, ROWBLK, D), lambda i: (0, i, 0)),
            pl.BlockSpec((NC, ROWBLK, D), lambda i: (0, i, 0)),
            pl.BlockSpec((1, ROWBLK, D), lambda i: (0, i, 0)),
            pl.BlockSpec((1, D), lambda i: (0, 0)),
        ],
        out_specs=pl.BlockSpec((ROWBLK, D), lambda i: (i, 0)),
        out_shape=jax.ShapeDtypeStruct((N, D), jnp.float32),
    )(acc2, deg, selft, b)


def kernel(features, edge_index, edge_types, W1, Wself1, b1, W2, Wself2, b2):
    src = edge_index[0]
    dst = edge_index[1]
    pad = EPAD - E
    src2d = jnp.concatenate([src, jnp.zeros((pad,), jnp.int32)]).reshape(-1, CHUNK)
    dst2d = jnp.concatenate([dst, jnp.full((pad,), N, jnp.int32)]).reshape(-1, CHUNK)
    et2d = jnp.concatenate([edge_types, jnp.zeros((pad,), jnp.int32)]).reshape(-1, CHUNK)
    wc1 = jnp.concatenate([W1, Wself1[None]], axis=0)
    wc2 = jnp.concatenate([W2, Wself2[None]], axis=0)

    deg = _sc_deg(dst2d)
    out1 = _tc_mm1(features, wc1)
    table1 = out1[:R].reshape(R * N, D)
    acc1 = _sc_scatter(table1, src2d, dst2d, et2d)
    out2 = _tc_mm2(acc1, deg, out1[R:], b1.reshape(1, D), wc2)
    table2 = out2[:R].reshape(R * N, D)
    acc2 = _sc_scatter(table2, src2d, dst2d, et2d)
    return _tc_final(acc2, deg, out2[R:], b2.reshape(1, D))


# speedup vs baseline: 10.0512x; 1.0827x over previous
import jax
import jax.numpy as jnp
from jax import lax
from jax.experimental import pallas as pl
from jax.experimental.pallas import tpu as pltpu
from jax.experimental.pallas import tpu_sc as plsc

N = 10000
E = 320000
R = 8
D = 128

NC = 2
NS = 16
NW = NC * NS
CHUNK = 128
CPW = 80
EPAD = NW * CPW * CHUNK
NPAD = 10112
RPT = NPAD // NS
GRP = 8

_MESH = plsc.VectorSubcoreMesh(core_axis_name="c", subcore_axis_name="s")


def _fill(ref, val16, nrows, width):
    def row(r, _):
        for k in range(width // 16):
            ref[r, pl.ds(k * 16, 16)] = val16
        return 0

    lax.fori_loop(0, nrows, row, 0)


def _zero_acc(acc_sh, stage, s):
    for t in range(5):
        sz = CHUNK if t < 4 else RPT - 4 * CHUNK
        pltpu.sync_copy(stage.at[pl.ds(0, sz)],
                        acc_sh.at[pl.ds(s * RPT + t * CHUNK, sz)])


def _writeback(acc_sh, stage, out, c, s):
    for t in range(5):
        sz = CHUNK if t < 4 else RPT - 4 * CHUNK
        pltpu.sync_copy(acc_sh.at[pl.ds(s * RPT + t * CHUNK, sz)],
                        stage.at[pl.ds(0, sz)])
        pltpu.sync_copy(stage.at[pl.ds(0, sz)],
                        out.at[pl.ds(c * NPAD + s * RPT + t * CHUNK, sz)])


def _sc_scatter(table, src2d, dst2d, et2d):
    scratch = (
        pltpu.VMEM_SHARED((NPAD, D), jnp.float32),
        pltpu.VMEM((GRP, CHUNK), jnp.int32),
        pltpu.VMEM((GRP, CHUNK), jnp.int32),
        pltpu.VMEM((GRP, CHUNK), jnp.int32),
        pltpu.VMEM((2, CHUNK), jnp.int32),
        pltpu.VMEM((CHUNK, D), jnp.float32),
        pltpu.VMEM((CHUNK, D), jnp.float32),
        pltpu.SemaphoreType.DMA,
        pltpu.SemaphoreType.DMA,
    )

    def body(table_hbm, src_hbm, dst_hbm, et_hbm, acc_out,
             acc_sh, src_v, dst_v, et_v, gkey_v, rows0, rows1, sem0, sem1):
        c = lax.axis_index("c")
        s = lax.axis_index("s")
        w = c * NS + s
        rows = (rows0, rows1)
        sems = (sem0, sem1)

        _fill(rows0, jnp.zeros((16,), jnp.float32), CHUNK, D)
        _zero_acc(acc_sh, rows0, s)
        plsc.subcore_barrier()

        def stage(g):
            base = w * CPW + g * GRP
            pltpu.sync_copy(src_hbm.at[pl.ds(base, GRP)], src_v)
            pltpu.sync_copy(dst_hbm.at[pl.ds(base, GRP)], dst_v)
            pltpu.sync_copy(et_hbm.at[pl.ds(base, GRP)], et_v)

        def keys(j, p):
            for i in range(CHUNK // 16):
                et16 = et_v[j, pl.ds(i * 16, 16)]
                src16 = src_v[j, pl.ds(i * 16, 16)]
                gkey_v[p, pl.ds(i * 16, 16)] = et16 * N + src16

        def fire(p):
            return pltpu.async_copy(table_hbm.at[gkey_v.at[p]], rows[p],
                                    sems[p])

        stage(0)
        keys(0, 0)
        fire(0)

        def drain_scatter(j, p):
            pltpu.make_async_copy(table_hbm.at[gkey_v.at[p]], rows[p],
                                  sems[p]).wait()
            pltpu.sync_copy(rows[p], acc_sh.at[dst_v.at[j]], add=True)

        def group(g, _):
            for j in range(GRP):
                p = j % 2
                nxt = (j + 1) % 2
                if j < GRP - 1:
                    keys(j + 1, nxt)
                    fire(nxt)
                    drain_scatter(j, p)
                else:
                    drain_scatter(j, p)

                    @pl.when(g < CPW // GRP - 1)
                    def _():
                        stage(g + 1)
                        keys(0, nxt)
                        fire(nxt)
            return 0

        lax.fori_loop(0, CPW // GRP, group, 0)
        plsc.subcore_barrier()
        _writeback(acc_sh, rows0, acc_out, c, s)

    out = pl.kernel(body,
                    out_type=jax.ShapeDtypeStruct((NC * NPAD, D), jnp.float32),
                    mesh=_MESH, scratch_types=scratch)(table, src2d, dst2d, et2d)
    return out.reshape(NC, NPAD, D)


def _sc_deg(dst2d):
    scratch = (
        pltpu.VMEM_SHARED((NPAD, D), jnp.float32),
        pltpu.VMEM((GRP, CHUNK), jnp.int32),
        pltpu.VMEM((CHUNK, D), jnp.float32),
    )

    def body(dst_hbm, deg_out, deg_sh, dst_v, ones_v):
        c = lax.axis_index("c")
        s = lax.axis_index("s")
        w = c * NS + s

        _fill(ones_v, jnp.zeros((16,), jnp.float32), CHUNK, D)
        _zero_acc(deg_sh, ones_v, s)
        _fill(ones_v, jnp.ones((16,), jnp.float32), CHUNK, D)
        plsc.subcore_barrier()

        def group(g, _):
            base = w * CPW + g * GRP
            pltpu.sync_copy(dst_hbm.at[pl.ds(base, GRP)], dst_v)
            for j in range(GRP):
                pltpu.sync_copy(ones_v, deg_sh.at[dst_v.at[j]], add=True)
            return 0

        lax.fori_loop(0, CPW // GRP, group, 0)
        plsc.subcore_barrier()
        _writeback(deg_sh, ones_v, deg_out, c, s)

    out = pl.kernel(body,
                    out_type=jax.ShapeDtypeStruct((NC * NPAD, D), jnp.float32),
                    mesh=_MESH, scratch_types=scratch)(dst2d)
    return out.reshape(NC, NPAD, D)


ROWBLK = 2000


def _mm1_body(x_ref, w_ref, o_ref):
    o_ref[0] = jnp.dot(x_ref[...], w_ref[0], preferred_element_type=jnp.float32)


def _tc_mm1(x, wc):
    grid = (wc.shape[0], N // ROWBLK)
    return pl.pallas_call(
        _mm1_body,
        grid=grid,
        in_specs=[
            pl.BlockSpec((ROWBLK, D), lambda j, i: (i, 0)),
            pl.BlockSpec((1, D, D), lambda j, i: (j, 0, 0)),
        ],
        out_specs=pl.BlockSpec((1, ROWBLK, D), lambda j, i: (j, i, 0)),
        out_shape=jax.ShapeDtypeStruct((wc.shape[0], N, D), jnp.float32),
    )(x, wc)


def _norm(acc_ref, deg_ref):
    a = acc_ref[0] + acc_ref[1]
    dg = deg_ref[0, :, 0:1] + deg_ref[1, :, 0:1]
    inv = 1.0 / jnp.maximum(dg, 1.0)
    return a * inv


def _mm2_body(acc_ref, deg_ref, self_ref, b_ref, w_ref, o_ref):
    h = _norm(acc_ref, deg_ref) + self_ref[0] + b_ref[...]
    h = jnp.maximum(h, 0.0)
    o_ref[0] = jnp.dot(h, w_ref[0], preferred_element_type=jnp.float32)


def _tc_mm2(acc, deg, selft, b, wc):
    grid = (wc.shape[0], N // ROWBLK)
    return pl.pallas_call(
        _mm2_body,
        grid=grid,
        in_specs=[
            pl.BlockSpec((NC, ROWBLK, D), lambda j, i: (0, i, 0)),
            pl.BlockSpec((NC, ROWBLK, D), lambda j, i: (0, i, 0)),
            pl.BlockSpec((1, ROWBLK, D), lambda j, i: (0, i, 0)),
            pl.BlockSpec((1, D), lambda j, i: (0, 0)),
            pl.BlockSpec((1, D, D), lambda j, i: (j, 0, 0)),
        ],
        out_specs=pl.BlockSpec((1, ROWBLK, D), lambda j, i: (j, i, 0)),
        out_shape=jax.ShapeDtypeStruct((wc.shape[0], N, D), jnp.float32),
    )(acc, deg, selft, b, wc)


def _final_body(acc_ref, deg_ref, self_ref, b_ref, o_ref):
    o_ref[...] = _norm(acc_ref, deg_ref) + self_ref[0] + b_ref[...]


def _tc_final(acc2, deg, selft, b):
    grid = (N // ROWBLK,)
    return pl.pallas_call(
        _final_body,
        grid=grid,
        in_specs=[
            pl.BlockSpec((NC, ROWBLK, D), lambda i: (0, i, 0)),
            pl.BlockSpec((NC, ROWBLK, D), lambda i: (0, i, 0)),
            pl.BlockSpec((1, ROWBLK, D), lambda i: (0, i, 0)),
            pl.BlockSpec((1, D), lambda i: (0, 0)),
        ],
        out_specs=pl.BlockSpec((ROWBLK, D), lambda i: (i, 0)),
        out_shape=jax.ShapeDtypeStruct((N, D), jnp.float32),
    )(acc2, deg, selft, b)


def kernel(features, edge_index, edge_types, W1, Wself1, b1, W2, Wself2, b2):
    src = edge_index[0]
    dst = edge_index[1]
    pad = EPAD - E
    src2d = jnp.concatenate([src, jnp.zeros((pad,), jnp.int32)]).reshape(-1, CHUNK)
    dst2d = jnp.concatenate([dst, jnp.full((pad,), N, jnp.int32)]).reshape(-1, CHUNK)
    et2d = jnp.concatenate([edge_types, jnp.zeros((pad,), jnp.int32)]).reshape(-1, CHUNK)
    wc1 = jnp.concatenate([W1, Wself1[None]], axis=0)
    wc2 = jnp.concatenate([W2, Wself2[None]], axis=0)

    deg = _sc_deg(dst2d)
    out1 = _tc_mm1(features, wc1)
    table1 = out1[:R].reshape(R * N, D)
    acc1 = _sc_scatter(table1, src2d, dst2d, et2d)
    out2 = _tc_mm2(acc1, deg, out1[R:], b1.reshape(1, D), wc2)
    table2 = out2[:R].reshape(R * N, D)
    acc2 = _sc_scatter(table2, src2d, dst2d, et2d)
    return _tc_final(acc2, deg, out2[R:], b2.reshape(1, D))


# speedup vs baseline: 10.3010x; 1.0248x over previous
import jax
import jax.numpy as jnp
from jax import lax
from jax.experimental import pallas as pl
from jax.experimental.pallas import tpu as pltpu
from jax.experimental.pallas import tpu_sc as plsc

N = 10000
E = 320000
R = 8
D = 128

NC = 2
NS = 16
NW = NC * NS
CHUNK = 128
CPW = 80
EPAD = NW * CPW * CHUNK
NPAD = 10112
RPT = NPAD // NS
GRP = 16

_MESH = plsc.VectorSubcoreMesh(core_axis_name="c", subcore_axis_name="s")


def _fill(ref, val16, nrows, width):
    def row(r, _):
        for k in range(width // 16):
            ref[r, pl.ds(k * 16, 16)] = val16
        return 0

    lax.fori_loop(0, nrows, row, 0)


def _zero_acc(acc_sh, stage, s):
    for t in range(5):
        sz = CHUNK if t < 4 else RPT - 4 * CHUNK
        pltpu.sync_copy(stage.at[pl.ds(0, sz)],
                        acc_sh.at[pl.ds(s * RPT + t * CHUNK, sz)])


def _writeback(acc_sh, stage, out, c, s):
    for t in range(5):
        sz = CHUNK if t < 4 else RPT - 4 * CHUNK
        pltpu.sync_copy(acc_sh.at[pl.ds(s * RPT + t * CHUNK, sz)],
                        stage.at[pl.ds(0, sz)])
        pltpu.sync_copy(stage.at[pl.ds(0, sz)],
                        out.at[pl.ds(c * NPAD + s * RPT + t * CHUNK, sz)])


def _sc_scatter(table, src2d, dst2d, et2d):
    scratch = (
        pltpu.VMEM_SHARED((NPAD, D), jnp.float32),
        pltpu.VMEM((GRP, CHUNK), jnp.int32),
        pltpu.VMEM((GRP, CHUNK), jnp.int32),
        pltpu.VMEM((GRP, CHUNK), jnp.int32),
        pltpu.VMEM((2, CHUNK), jnp.int32),
        pltpu.VMEM((CHUNK, D), jnp.float32),
        pltpu.VMEM((CHUNK, D), jnp.float32),
        pltpu.SemaphoreType.DMA,
        pltpu.SemaphoreType.DMA,
        pltpu.SemaphoreType.DMA,
        pltpu.SemaphoreType.DMA,
    )

    def body(table_hbm, src_hbm, dst_hbm, et_hbm, acc_out,
             acc_sh, src_v, dst_v, et_v, gkey_v, rows0, rows1,
             sem0, sem1, ssem0, ssem1):
        c = lax.axis_index("c")
        s = lax.axis_index("s")
        w = c * NS + s
        rows = (rows0, rows1)
        sems = (sem0, sem1)
        ssems = (ssem0, ssem1)

        _fill(rows0, jnp.zeros((16,), jnp.float32), CHUNK, D)
        _zero_acc(acc_sh, rows0, s)
        plsc.subcore_barrier()

        def stage(g):
            base = w * CPW + g * GRP
            pltpu.sync_copy(src_hbm.at[pl.ds(base, GRP)], src_v)
            pltpu.sync_copy(dst_hbm.at[pl.ds(base, GRP)], dst_v)
            pltpu.sync_copy(et_hbm.at[pl.ds(base, GRP)], et_v)

        def keys(j, p):
            for i in range(CHUNK // 16):
                et16 = et_v[j, pl.ds(i * 16, 16)]
                src16 = src_v[j, pl.ds(i * 16, 16)]
                gkey_v[p, pl.ds(i * 16, 16)] = et16 * N + src16

        def fire(p):
            return pltpu.async_copy(table_hbm.at[gkey_v.at[p]], rows[p],
                                    sems[p])

        stage(0)
        keys(0, 0)
        fire(0)

        def drain_gather_fire_scatter(j, p):
            pltpu.make_async_copy(table_hbm.at[gkey_v.at[p]], rows[p],
                                  sems[p]).wait()
            pltpu.async_copy(rows[p], acc_sh.at[dst_v.at[j]], ssems[p],
                             add=True)

        def wait_scatter(p):
            pltpu.make_async_copy(rows[p], acc_sh.at[dst_v.at[0]],
                                  ssems[p]).wait()

        def group(g, _):
            for j in range(GRP):
                p = j % 2
                nxt = (j + 1) % 2
                if j < GRP - 1:
                    keys(j + 1, nxt)
                    if j > 0:
                        wait_scatter(nxt)
                    else:
                        @pl.when(g > 0)
                        def _():
                            wait_scatter(nxt)
                    fire(nxt)
                    drain_gather_fire_scatter(j, p)
                else:
                    drain_gather_fire_scatter(j, p)

                    @pl.when(g < CPW // GRP - 1)
                    def _():
                        stage(g + 1)
                        keys(0, nxt)
                        wait_scatter(nxt)
                        fire(nxt)
            return 0

        lax.fori_loop(0, CPW // GRP, group, 0)
        wait_scatter(0)
        wait_scatter(1)
        plsc.subcore_barrier()
        _writeback(acc_sh, rows0, acc_out, c, s)

    out = pl.kernel(body,
                    out_type=jax.ShapeDtypeStruct((NC * NPAD, D), jnp.float32),
                    mesh=_MESH, scratch_types=scratch)(table, src2d, dst2d, et2d)
    return out.reshape(NC, NPAD, D)


def _sc_deg(dst2d):
    scratch = (
        pltpu.VMEM_SHARED((NPAD, D), jnp.float32),
        pltpu.VMEM((GRP, CHUNK), jnp.int32),
        pltpu.VMEM((CHUNK, D), jnp.float32),
    )

    def body(dst_hbm, deg_out, deg_sh, dst_v, ones_v):
        c = lax.axis_index("c")
        s = lax.axis_index("s")
        w = c * NS + s

        _fill(ones_v, jnp.zeros((16,), jnp.float32), CHUNK, D)
        _zero_acc(deg_sh, ones_v, s)
        _fill(ones_v, jnp.ones((16,), jnp.float32), CHUNK, D)
        plsc.subcore_barrier()

        def group(g, _):
            base = w * CPW + g * GRP
            pltpu.sync_copy(dst_hbm.at[pl.ds(base, GRP)], dst_v)
            for j in range(GRP):
                pltpu.sync_copy(ones_v, deg_sh.at[dst_v.at[j]], add=True)
            return 0

        lax.fori_loop(0, CPW // GRP, group, 0)
        plsc.subcore_barrier()
        _writeback(deg_sh, ones_v, deg_out, c, s)

    out = pl.kernel(body,
                    out_type=jax.ShapeDtypeStruct((NC * NPAD, D), jnp.float32),
                    mesh=_MESH, scratch_types=scratch)(dst2d)
    return out.reshape(NC, NPAD, D)


ROWBLK = 2000


def _mm1_body(x_ref, w_ref, o_ref):
    o_ref[0] = jnp.dot(x_ref[...], w_ref[0], preferred_element_type=jnp.float32)


def _tc_mm1(x, wc):
    grid = (wc.shape[0], N // ROWBLK)
    return pl.pallas_call(
        _mm1_body,
        grid=grid,
        in_specs=[
            pl.BlockSpec((ROWBLK, D), lambda j, i: (i, 0)),
            pl.BlockSpec((1, D, D), lambda j, i: (j, 0, 0)),
        ],
        out_specs=pl.BlockSpec((1, ROWBLK, D), lambda j, i: (j, i, 0)),
        out_shape=jax.ShapeDtypeStruct((wc.shape[0], N, D), jnp.float32),
    )(x, wc)


def _norm(acc_ref, deg_ref):
    a = acc_ref[0] + acc_ref[1]
    dg = deg_ref[0, :, 0:1] + deg_ref[1, :, 0:1]
    inv = 1.0 / jnp.maximum(dg, 1.0)
    return a * inv


def _mm2_body(acc_ref, deg_ref, self_ref, b_ref, w_ref, o_ref):
    h = _norm(acc_ref, deg_ref) + self_ref[0] + b_ref[...]
    h = jnp.maximum(h, 0.0)
    o_ref[0] = jnp.dot(h, w_ref[0], preferred_element_type=jnp.float32)


def _tc_mm2(acc, deg, selft, b, wc):
    grid = (wc.shape[0], N // ROWBLK)
    return pl.pallas_call(
        _mm2_body,
        grid=grid,
        in_specs=[
            pl.BlockSpec((NC, ROWBLK, D), lambda j, i: (0, i, 0)),
            pl.BlockSpec((NC, ROWBLK, D), lambda j, i: (0, i, 0)),
            pl.BlockSpec((1, ROWBLK, D), lambda j, i: (0, i, 0)),
            pl.BlockSpec((1, D), lambda j, i: (0, 0)),
            pl.BlockSpec((1, D, D), lambda j, i: (j, 0, 0)),
        ],
        out_specs=pl.BlockSpec((1, ROWBLK, D), lambda j, i: (j, i, 0)),
        out_shape=jax.ShapeDtypeStruct((wc.shape[0], N, D), jnp.float32),
    )(acc, deg, selft, b, wc)


def _final_body(acc_ref, deg_ref, self_ref, b_ref, o_ref):
    o_ref[...] = _norm(acc_ref, deg_ref) + self_ref[0] + b_ref[...]


def _tc_final(acc2, deg, selft, b):
    grid = (N // ROWBLK,)
    return pl.pallas_call(
        _final_body,
        grid=grid,
        in_specs=[
            pl.BlockSpec((NC, ROWBLK, D), lambda i: (0, i, 0)),
            pl.BlockSpec((NC, ROWBLK, D), lambda i: (0, i, 0)),
            pl.BlockSpec((1, ROWBLK, D), lambda i: (0, i, 0)),
            pl.BlockSpec((1, D), lambda i: (0, 0)),
        ],
        out_specs=pl.BlockSpec((ROWBLK, D), lambda i: (i, 0)),
        out_shape=jax.ShapeDtypeStruct((N, D), jnp.float32),
    )(acc2, deg, selft, b)


def kernel(features, edge_index, edge_types, W1, Wself1, b1, W2, Wself2, b2):
    src = edge_index[0]
    dst = edge_index[1]
    pad = EPAD - E
    src2d = jnp.concatenate([src, jnp.zeros((pad,), jnp.int32)]).reshape(-1, CHUNK)
    dst2d = jnp.concatenate([dst, jnp.full((pad,), N, jnp.int32)]).reshape(-1, CHUNK)
    et2d = jnp.concatenate([edge_types, jnp.zeros((pad,), jnp.int32)]).reshape(-1, CHUNK)
    wc1 = jnp.concatenate([W1, Wself1[None]], axis=0)
    wc2 = jnp.concatenate([W2, Wself2[None]], axis=0)

    deg = _sc_deg(dst2d)
    out1 = _tc_mm1(features, wc1)
    table1 = out1[:R].reshape(R * N, D)
    acc1 = _sc_scatter(table1, src2d, dst2d, et2d)
    out2 = _tc_mm2(acc1, deg, out1[R:], b1.reshape(1, D), wc2)
    table2 = out2[:R].reshape(R * N, D)
    acc2 = _sc_scatter(table2, src2d, dst2d, et2d)
    return _tc_final(acc2, deg, out2[R:], b2.reshape(1, D))
